# Initial kernel scaffold; baseline (speedup 1.0000x reference)
import jax
import jax.numpy as jnp
from jax.experimental import pallas as pl


def kernel(x, emb_table, fc_w, fc_b):
    raise NotImplementedError("write your pallas kernel here")


# speedup vs baseline: 3.1758x; 3.1758x over previous
import functools

import jax
import jax.numpy as jnp
from jax.experimental import pallas as pl
from jax.experimental.pallas import tpu as pltpu

_NUM_FEATS = 3
_ROW_W = 1 + _NUM_FEATS


def _embed_kernel(x_ref, r_ref, t_ref, o_ref, *, v):
    xb = x_ref[...]
    m = jnp.dot(xb, r_ref[...], preferred_element_type=jnp.float32)
    m_id = m[:, :v]
    m_ft = m[:, v:]
    lane = jax.lax.broadcasted_iota(jnp.int32, (1, v), 1).astype(jnp.float32)
    onehot = (jnp.abs(m_id - lane) < 0.5).astype(jnp.float32)
    o_ref[...] = m_ft + jnp.dot(onehot, t_ref[...],
                                preferred_element_type=jnp.float32)


def kernel(x, emb_table, fc_w, fc_b, *, tile_rows=4096):
    n = x.shape[0]
    v, d = emb_table.shape

    w_t = fc_w.T
    t_tab = jnp.dot(emb_table, w_t[:d],
                    precision=jax.lax.Precision.HIGHEST) + fc_b
    r_id = jnp.concatenate(
        [jnp.ones((1, v), jnp.float32), jnp.zeros((_NUM_FEATS, v), jnp.float32)],
        axis=0)
    r_ft = jnp.concatenate(
        [jnp.zeros((1, d), jnp.float32), w_t[d:]], axis=0)
    r = jnp.concatenate([r_id, r_ft], axis=1)

    tm = max(8, min(tile_rows, ((n + 7) // 8) * 8))
    tm -= tm % 8
    n_pad = pl.cdiv(n, tm) * tm
    if n_pad != n:
        x = jnp.pad(x, ((0, n_pad - n), (0, 0)))

    out = pl.pallas_call(
        functools.partial(_embed_kernel, v=v),
        out_shape=jax.ShapeDtypeStruct((n_pad, d), jnp.float32),
        grid=(n_pad // tm,),
        in_specs=[
            pl.BlockSpec((tm, _ROW_W), lambda i: (i, 0)),
            pl.BlockSpec((_ROW_W, v + d), lambda i: (0, 0)),
            pl.BlockSpec((v, d), lambda i: (0, 0)),
        ],
        out_specs=pl.BlockSpec((tm, d), lambda i: (i, 0)),
        compiler_params=pltpu.CompilerParams(
            dimension_semantics=("parallel",)),
    )(x, r, t_tab)

    return out[:n]


# speedup vs baseline: 6.8180x; 2.1469x over previous
import functools

import jax
import jax.numpy as jnp
from jax.experimental import pallas as pl
from jax.experimental.pallas import tpu as pltpu

_NUM_FEATS = 3
_ROW_W = 1 + _NUM_FEATS


def _embed_kernel(xt_ref, r_ref, t_ref, o_ref, *, v):
    xb = xt_ref[...]
    m = jax.lax.dot_general(
        xb, r_ref[...],
        dimension_numbers=(((0,), (0,)), ((), ())),
        preferred_element_type=jnp.float32)
    m_id = m[:, :v]
    m_ft = m[:, v:]
    lane = jax.lax.broadcasted_iota(jnp.int32, (1, v), 1).astype(jnp.float32)
    onehot = (jnp.abs(m_id - lane) < 0.5).astype(jnp.float32)
    o_ref[...] = m_ft + jnp.dot(onehot, t_ref[...],
                                preferred_element_type=jnp.float32)


def kernel(x, emb_table, fc_w, fc_b, *, tile_rows=4096):
    n = x.shape[0]
    v, d = emb_table.shape

    w_t = fc_w.T
    t_tab = jnp.dot(emb_table, w_t[:d],
                    precision=jax.lax.Precision.HIGHEST) + fc_b
    r_id = jnp.concatenate(
        [jnp.ones((1, v), jnp.float32), jnp.zeros((_NUM_FEATS, v), jnp.float32)],
        axis=0)
    r_ft = jnp.concatenate(
        [jnp.zeros((1, d), jnp.float32), w_t[d:]], axis=0)
    r = jnp.concatenate([r_id, r_ft], axis=1)

    tm = max(128, min(tile_rows, ((n + 127) // 128) * 128))
    tm -= tm % 128
    n_pad = pl.cdiv(n, tm) * tm
    if n_pad != n:
        x = jnp.pad(x, ((0, n_pad - n), (0, 0)))
    xt = x.T

    out = pl.pallas_call(
        functools.partial(_embed_kernel, v=v),
        out_shape=jax.ShapeDtypeStruct((n_pad, d), jnp.float32),
        grid=(n_pad // tm,),
        in_specs=[
            pl.BlockSpec((_ROW_W, tm), lambda i: (0, i)),
            pl.BlockSpec((_ROW_W, v + d), lambda i: (0, 0)),
            pl.BlockSpec((v, d), lambda i: (0, 0)),
        ],
        out_specs=pl.BlockSpec((tm, d), lambda i: (i, 0)),
        compiler_params=pltpu.CompilerParams(
            dimension_semantics=("parallel",)),
    )(xt, r, t_tab)

    return out[:n]
